# baseline (speedup 1.0000x reference)
import functools

import jax
import jax.numpy as jnp
from jax import lax
from jax.experimental import pallas as pl
from jax.experimental.pallas import tpu as pltpu

BATCH = 32
NT = 2048
TEXT_DIM = 512
VPAD = 1024
BLK = 512
R = BATCH * NT
GRID = R // BLK


def _tc_body(seq_ref, idx_ref, table_ref, out_ref):
    pid = pl.program_id(0)
    idx = idx_ref[0, 0, :]
    col = (lax.broadcasted_iota(jnp.int32, (BLK, 1), 0) + pid * BLK) & (NT - 1)
    t = jnp.where(col < seq_ref[0], idx[:, None] + 1, 0)
    vocab = lax.broadcasted_iota(jnp.int32, (BLK, VPAD), 1)
    onehot = (t == vocab).astype(jnp.bfloat16)
    out_ref[...] = jnp.dot(onehot, table_ref[...],
                           preferred_element_type=jnp.float32)


def _tc_embed(text_flat, seq_len_arr, table_pad):
    return pl.pallas_call(
        _tc_body,
        grid=(GRID,),
        in_specs=[
            pl.BlockSpec(memory_space=pltpu.SMEM),
            pl.BlockSpec((1, 1, BLK), lambda i: (i, 0, 0)),
            pl.BlockSpec((VPAD, TEXT_DIM), lambda i: (0, 0)),
        ],
        out_specs=pl.BlockSpec((BLK, TEXT_DIM), lambda i: (i, 0)),
        out_shape=jax.ShapeDtypeStruct((R, TEXT_DIM), jnp.float32),
    )(seq_len_arr, text_flat, table_pad)


def kernel(text, seq_len, text_embed_weight):
    text2d = text.astype(jnp.int32).reshape(GRID, 1, BLK)
    slv = jnp.asarray([seq_len], dtype=jnp.int32)
    table_pad = jnp.zeros((VPAD, TEXT_DIM), jnp.bfloat16).at[:1001].set(
        text_embed_weight.astype(jnp.bfloat16))
    out = _tc_embed(text2d, slv, table_pad)
    return out.reshape(BATCH, NT, TEXT_DIM)
